# baseline (speedup 1.0000x reference)
import functools

import numpy as np
import jax
import jax.numpy as jnp
from jax import lax
from jax.experimental import pallas as pl
from jax.experimental.pallas import tpu as pltpu

_NC = 10
_PAIRS = [(i, j) for i in range(_NC) for j in range(i + 1, _NC)]
_NE = len(_PAIRS)

_C1 = np.array([p[0] for p in _PAIRS])
_C2 = np.array([p[1] for p in _PAIRS])


def _sel_mats():
    ea0 = np.zeros((2 * _NC * _NC, _NE), np.float32)
    ea1 = np.zeros_like(ea0)
    ev0 = np.zeros_like(ea0)
    ev1 = np.zeros_like(ea0)
    oc1 = np.zeros((_NC, _NE), np.float32)
    oc2 = np.zeros_like(oc1)
    for p in range(_NE):
        c1, c2 = int(_C1[p]), int(_C2[p])
        ea0[(2 * c1) * _NC + c2, p] = 1.0
        ea1[(2 * c1 + 1) * _NC + c2, p] = 1.0
        ev0[(2 * c2) * _NC + c1, p] = 1.0
        ev1[(2 * c2 + 1) * _NC + c1, p] = 1.0
        oc1[c1, p] = 1.0
        oc2[c2, p] = 1.0
    return ea0, ea1, ev0, ev1, oc1, oc2


_EA0, _EA1, _EV0, _EV1, _OC1, _OC2 = _sel_mats()

_CP = functools.partial(pltpu.CompilerParams,
                        vmem_limit_bytes=56 * 1024 * 1024)


def _pool2(y):
    h = (y.shape[-3] // 2) * 2
    w = (y.shape[-2] // 2) * 2
    y = y[..., :h, :w, :]
    y = y.reshape(y.shape[:-3] + (h // 2, 2, w // 2, 2, y.shape[-1]))
    return jnp.max(y, axis=(-4, -2))


def _s2d(x):
    sh = x.shape
    h, w, c = sh[-3] // 2, sh[-2] // 2, sh[-1]
    y = x.reshape(sh[:-3] + (h, 2, w, 2, c))
    nd = len(sh) - 3
    perm = tuple(range(nd)) + (nd, nd + 2, nd + 1, nd + 3, nd + 4)
    return y.transpose(perm).reshape(sh[:-3] + (h, w, 4 * c))


def _conv1_body(p_ref, w_ref, b_ref, o_ref):
    x = p_ref[0]
    h, w, k = x.shape
    y = x.reshape(h * w, k) @ w_ref[...] + b_ref[...]
    y = jnp.maximum(y, 0.0).reshape(h, w, w_ref.shape[1])
    o_ref[0] = _pool2(y)


def _conv1(patches, wmat, bias, row_chunks):
    n, h, w, k = patches.shape
    co = wmat.shape[1]
    hc = h // row_chunks
    return pl.pallas_call(
        _conv1_body,
        grid=(n, row_chunks),
        in_specs=[
            pl.BlockSpec((1, hc, w, k), lambda i, r: (i, r, 0, 0)),
            pl.BlockSpec((k, co), lambda i, r: (0, 0)),
            pl.BlockSpec((1, co), lambda i, r: (0, 0)),
        ],
        out_specs=pl.BlockSpec((1, hc // 2, w // 2, co),
                               lambda i, r: (i, r, 0, 0)),
        out_shape=jax.ShapeDtypeStruct((n, h // 2, w // 2, co), jnp.float32),
        compiler_params=_CP(dimension_semantics=("parallel", "arbitrary")),
    )(patches, wmat, bias)


def _mconv_body(x_ref, w_ref, b_ref, o_ref, *, na, nb, oh, ow):
    x = x_ref[0]
    n = x.shape[0]
    acc = None
    idx = 0
    for oa in range(na):
        for ob in range(nb):
            xs = x[:, oa:oa + oh, ob:ob + ow, :].reshape(n * oh * ow, -1)
            t = xs @ w_ref[0, idx]
            acc = t if acc is None else acc + t
            idx += 1
    y = jnp.maximum(acc + b_ref[0], 0.0)
    o_ref[0] = _pool2(y.reshape(n, oh, ow, -1))


def _member_conv(x, wmat, bias, na, nb, oh, ow, nc=None):
    ne, n, hp, wp, ci4 = x.shape
    noff, k4, co = wmat.shape[1], wmat.shape[2], wmat.shape[3]
    nc = n if nc is None else nc
    body = functools.partial(_mconv_body, na=na, nb=nb, oh=oh, ow=ow)
    return pl.pallas_call(
        body,
        grid=(ne, n // nc),
        in_specs=[
            pl.BlockSpec((1, nc, hp, wp, ci4), lambda m, f: (m, f, 0, 0, 0)),
            pl.BlockSpec((1, noff, k4, co), lambda m, f: (m, 0, 0, 0)),
            pl.BlockSpec((1, 1, co), lambda m, f: (m, 0, 0)),
        ],
        out_specs=pl.BlockSpec((1, nc, oh // 2, ow // 2, co),
                               lambda m, f: (m, f, 0, 0, 0)),
        out_shape=jax.ShapeDtypeStruct((ne, n, oh // 2, ow // 2, co),
                                       jnp.float32),
        compiler_params=_CP(dimension_semantics=("parallel", "parallel")),
    )(x, wmat, bias)


def _w_off(w):
    ne, co, ci, kh, kw = w.shape
    na, nb = (kh + 1) // 2, (kw + 1) // 2
    wo = jnp.zeros((ne, na, nb, 4, ci, co), w.dtype)
    for oa in range(na):
        for ob in range(nb):
            for r1 in range(2):
                for r2 in range(2):
                    th, tw = 2 * oa + r1, 2 * ob + r2
                    if th < kh and tw < kw:
                        wo = wo.at[:, oa, ob, r1 * 2 + r2].set(
                            w[:, :, :, th, tw].transpose(0, 2, 1))
    return wo.reshape(ne, na * nb, 4 * ci, co)


def _lstm_body(x_ref, wi0_ref, wh0_ref, b0_ref, wi1_ref, wh1_ref, b1_ref,
               fw_ref, fb_ref, o_ref, *, b, t, hid):
    x = x_ref[0].reshape(b * t, -1)

    def run(seq_flat, wi, bias, wh):
        xw = (seq_flat @ wi + bias).reshape(b, t, 4 * hid)
        h = jnp.zeros((b, hid), jnp.float32)
        c = jnp.zeros((b, hid), jnp.float32)
        hs = []
        for step in range(t):
            g = xw[:, step, :] + h @ wh
            ig = jax.nn.sigmoid(g[:, :hid])
            fg = jax.nn.sigmoid(g[:, hid:2 * hid])
            gg = jnp.tanh(g[:, 2 * hid:3 * hid])
            og = jax.nn.sigmoid(g[:, 3 * hid:])
            c = fg * c + ig * gg
            h = og * jnp.tanh(c)
            hs.append(h)
        return hs

    hs0 = run(x, wi0_ref[0], b0_ref[0], wh0_ref[0])
    h1in = jnp.stack(hs0, axis=1).reshape(b * t, hid)
    hs1 = run(h1in, wi1_ref[0], b1_ref[0], wh1_ref[0])
    o_ref[0] = hs1[-1] @ fw_ref[0] + fb_ref[0]


def _lstm_fc(x, p, b, t, i):
    hid = 128
    wi0 = p['l0_wih'].transpose(0, 2, 1)
    wh0 = p['l0_whh'].transpose(0, 2, 1)
    b0 = (p['l0_bih'] + p['l0_bhh'])[:, None, :]
    wi1 = p['l1_wih'].transpose(0, 2, 1)
    wh1 = p['l1_whh'].transpose(0, 2, 1)
    b1 = (p['l1_bih'] + p['l1_bhh'])[:, None, :]
    fw = p['fc_w'].transpose(0, 2, 1)
    fb = p['fc_b'][:, None, :]
    body = functools.partial(_lstm_body, b=b, t=t, hid=hid)
    full = lambda a: pl.BlockSpec((1,) + a.shape[1:],
                                  lambda m: (m,) + (0,) * (a.ndim - 1))
    ins = (x, wi0, wh0, b0, wi1, wh1, b1, fw, fb)
    return pl.pallas_call(
        body,
        grid=(_NE,),
        in_specs=[full(a) for a in ins],
        out_specs=pl.BlockSpec((1, b, 2), lambda m: (m, 0, 0)),
        out_shape=jax.ShapeDtypeStruct((_NE, b, 2), jnp.float32),
        compiler_params=_CP(dimension_semantics=("parallel",)),
    )(*ins)


def _asm_body(a_ref, v_ref, ea0_ref, ea1_ref, ev0_ref, ev1_ref,
              oc1_ref, oc2_ref, votes_ref, mat_ref):
    a = a_ref[...]
    v = v_ref[...]
    a0, a1 = a[:, :, 0], a[:, :, 1]
    v0, v1 = v[:, :, 0], v[:, :, 1]
    mat_ref[...] = (ea0_ref[...] @ a0 + ea1_ref[...] @ a1
                    + ev0_ref[...] @ v1 + ev1_ref[...] @ v0)
    pa = (a0 >= a1).astype(jnp.float32)
    pv = (v0 >= v1).astype(jnp.float32)
    votes_ref[...] = (oc1_ref[...] @ (pa + pv)
                      + oc2_ref[...] @ (2.0 - pa - pv))


def _assemble(out_a, out_v):
    bsz = out_a.shape[1]
    consts = tuple(jnp.asarray(m) for m in
                   (_EA0, _EA1, _EV0, _EV1, _OC1, _OC2))
    ins = (out_a, out_v) + consts
    full = lambda a: pl.BlockSpec(a.shape, lambda i: (0,) * a.ndim)
    votes_t, mat_t = pl.pallas_call(
        _asm_body,
        grid=(1,),
        in_specs=[full(a) for a in ins],
        out_specs=[full(jnp.zeros((_NC, bsz))),
                   full(jnp.zeros((2 * _NC * _NC, bsz)))],
        out_shape=[jax.ShapeDtypeStruct((_NC, bsz), jnp.float32),
                   jax.ShapeDtypeStruct((2 * _NC * _NC, bsz), jnp.float32)],
        compiler_params=_CP(dimension_semantics=("arbitrary",)),
    )(*ins)
    return votes_t.T, mat_t.T.reshape(bsz, 2 * _NC, _NC)


def kernel(x_audio, x_video, audio_params, video_params):
    ap, vp = audio_params, video_params
    bsz = x_audio.shape[0]

    pa = lax.conv_general_dilated_patches(
        x_audio.reshape(bsz, 128, 431, 1), (11, 11), (2, 2),
        [(1, 1), (1, 1)], dimension_numbers=('NHWC', 'HWIO', 'NHWC'))
    wa1 = ap['c1w'].reshape(_NE * 16, 121).T
    ba1 = ap['c1b'].reshape(1, _NE * 16)
    a1 = _conv1(pa, wa1, ba1, row_chunks=3)
    a1 = jnp.pad(a1, ((0, 0), (1, 1), (1, 1), (0, 0)))
    a1 = (a1.reshape(bsz, 16, 2, 54, 2, _NE, 16)
          .transpose(5, 0, 1, 3, 2, 4, 6)
          .reshape(_NE, bsz, 16, 54, 64))
    a2 = _member_conv(a1, _w_off(ap['c2w']), ap['c2b'][:, None, :],
                      na=4, nb=4, oh=13, ow=51, nc=4)
    a2 = jnp.pad(a2, ((0, 0), (0, 0), (1, 1), (1, 2), (0, 0)))
    a3 = _member_conv(_s2d(a2), _w_off(ap['c3w']), ap['c3b'][:, None, :],
                      na=3, nb=3, oh=2, ow=12)
    out_a = _lstm_fc(a3.reshape(_NE, bsz, 6, 48), ap, b=bsz, t=6, i=48)

    tlen = x_video.shape[1]
    xv = x_video.reshape((bsz * tlen,) + x_video.shape[2:])
    pv = lax.conv_general_dilated_patches(
        xv.transpose(0, 2, 3, 1), (7, 7), (3, 3),
        [(1, 1), (1, 1)], dimension_numbers=('NHWC', 'HWIO', 'NHWC'))
    wv1 = vp['c1w'].reshape(_NE * 16, 147).T
    bv1 = vp['c1b'].reshape(1, _NE * 16)
    v1 = _conv1(pv, wv1, bv1, row_chunks=1)
    v1 = jnp.pad(v1, ((0, 0), (1, 2), (1, 2), (0, 0)))
    v1 = (v1.reshape(bsz * tlen, 20, 2, 20, 2, _NE, 16)
          .transpose(5, 0, 1, 3, 2, 4, 6)
          .reshape(_NE, bsz * tlen, 20, 20, 64))
    v2 = _member_conv(v1, _w_off(vp['c2w']), vp['c2b'][:, None, :],
                      na=3, nb=3, oh=18, ow=18, nc=16)
    v2 = jnp.pad(v2, ((0, 0), (0, 0), (1, 2), (1, 2), (0, 0)))
    v3 = _member_conv(_s2d(v2), _w_off(vp['c3w']), vp['c3b'][:, None, :],
                      na=3, nb=3, oh=4, ow=4)
    xv_seq = v3.transpose(0, 1, 4, 2, 3).reshape(_NE, bsz, tlen, 128)
    out_v = _lstm_fc(xv_seq, vp, b=bsz, t=tlen, i=128)

    return _assemble(out_a, out_v)


# speedup vs baseline: 1.0048x; 1.0048x over previous
import functools

import numpy as np
import jax
import jax.numpy as jnp
from jax import lax
from jax.experimental import pallas as pl
from jax.experimental.pallas import tpu as pltpu

_NC = 10
_PAIRS = [(i, j) for i in range(_NC) for j in range(i + 1, _NC)]
_NE = len(_PAIRS)

_C1 = np.array([p[0] for p in _PAIRS])
_C2 = np.array([p[1] for p in _PAIRS])


def _sel_mats():
    ea0 = np.zeros((2 * _NC * _NC, _NE), np.float32)
    ea1 = np.zeros_like(ea0)
    ev0 = np.zeros_like(ea0)
    ev1 = np.zeros_like(ea0)
    oc1 = np.zeros((_NC, _NE), np.float32)
    oc2 = np.zeros_like(oc1)
    for p in range(_NE):
        c1, c2 = int(_C1[p]), int(_C2[p])
        ea0[(2 * c1) * _NC + c2, p] = 1.0
        ea1[(2 * c1 + 1) * _NC + c2, p] = 1.0
        ev0[(2 * c2) * _NC + c1, p] = 1.0
        ev1[(2 * c2 + 1) * _NC + c1, p] = 1.0
        oc1[c1, p] = 1.0
        oc2[c2, p] = 1.0
    return ea0, ea1, ev0, ev1, oc1, oc2


_EA0, _EA1, _EV0, _EV1, _OC1, _OC2 = _sel_mats()

_CP = functools.partial(pltpu.CompilerParams,
                        vmem_limit_bytes=56 * 1024 * 1024)


def _pool2(y):
    h = (y.shape[-3] // 2) * 2
    w = (y.shape[-2] // 2) * 2
    y = y[..., :h, :w, :]
    y = y.reshape(y.shape[:-3] + (h // 2, 2, w // 2, 2, y.shape[-1]))
    return jnp.max(y, axis=(-4, -2))


def _s2d(x):
    sh = x.shape
    h, w, c = sh[-3] // 2, sh[-2] // 2, sh[-1]
    y = x.reshape(sh[:-3] + (h, 2, w, 2, c))
    nd = len(sh) - 3
    perm = tuple(range(nd)) + (nd, nd + 2, nd + 1, nd + 3, nd + 4)
    return y.transpose(perm).reshape(sh[:-3] + (h, w, 4 * c))


def _conv1_body(p_ref, w_ref, b_ref, o_ref):
    x = p_ref[0]
    h, w, k = x.shape
    y = x.reshape(h * w, k) @ w_ref[...] + b_ref[...]
    y = jnp.maximum(y, 0.0).reshape(h, w, w_ref.shape[1])
    o_ref[0] = _pool2(y)


def _conv1(patches, wmat, bias, row_chunks):
    n, h, w, k = patches.shape
    co = wmat.shape[1]
    hc = h // row_chunks
    return pl.pallas_call(
        _conv1_body,
        grid=(n, row_chunks),
        in_specs=[
            pl.BlockSpec((1, hc, w, k), lambda i, r: (i, r, 0, 0)),
            pl.BlockSpec((k, co), lambda i, r: (0, 0)),
            pl.BlockSpec((1, co), lambda i, r: (0, 0)),
        ],
        out_specs=pl.BlockSpec((1, hc // 2, w // 2, co),
                               lambda i, r: (i, r, 0, 0)),
        out_shape=jax.ShapeDtypeStruct((n, h // 2, w // 2, co), jnp.float32),
        compiler_params=_CP(dimension_semantics=("parallel", "arbitrary")),
    )(patches, wmat, bias)


def _mconv_body(x_ref, w_ref, b_ref, o_ref, *, na, nb, oh, ow):
    x = x_ref[0]
    n = x.shape[0]
    acc = None
    idx = 0
    for oa in range(na):
        for ob in range(nb):
            xs = x[:, oa:oa + oh, ob:ob + ow, :].reshape(n * oh * ow, -1)
            t = xs @ w_ref[0, idx]
            acc = t if acc is None else acc + t
            idx += 1
    y = jnp.maximum(acc + b_ref[0], 0.0)
    o_ref[0] = _pool2(y.reshape(n, oh, ow, -1))


def _member_conv(x, wmat, bias, na, nb, oh, ow, nc=None):
    ne, n, hp, wp, ci4 = x.shape
    noff, k4, co = wmat.shape[1], wmat.shape[2], wmat.shape[3]
    nc = n if nc is None else nc
    body = functools.partial(_mconv_body, na=na, nb=nb, oh=oh, ow=ow)
    return pl.pallas_call(
        body,
        grid=(ne, n // nc),
        in_specs=[
            pl.BlockSpec((1, nc, hp, wp, ci4), lambda m, f: (m, f, 0, 0, 0)),
            pl.BlockSpec((1, noff, k4, co), lambda m, f: (m, 0, 0, 0)),
            pl.BlockSpec((1, 1, co), lambda m, f: (m, 0, 0)),
        ],
        out_specs=pl.BlockSpec((1, nc, oh // 2, ow // 2, co),
                               lambda m, f: (m, f, 0, 0, 0)),
        out_shape=jax.ShapeDtypeStruct((ne, n, oh // 2, ow // 2, co),
                                       jnp.float32),
        compiler_params=_CP(dimension_semantics=("parallel", "parallel")),
    )(x, wmat, bias)


def _w_off(w):
    ne, co, ci, kh, kw = w.shape
    na, nb = (kh + 1) // 2, (kw + 1) // 2
    wo = jnp.zeros((ne, na, nb, 4, ci, co), w.dtype)
    for oa in range(na):
        for ob in range(nb):
            for r1 in range(2):
                for r2 in range(2):
                    th, tw = 2 * oa + r1, 2 * ob + r2
                    if th < kh and tw < kw:
                        wo = wo.at[:, oa, ob, r1 * 2 + r2].set(
                            w[:, :, :, th, tw].transpose(0, 2, 1))
    return wo.reshape(ne, na * nb, 4 * ci, co)


def _lstm_body(x_ref, wi0_ref, wh0_ref, b0_ref, wi1_ref, wh1_ref, b1_ref,
               fw_ref, fb_ref, o_ref, *, b, t, hid):
    x = x_ref[0].reshape(b * t, -1)

    def run(seq_flat, wi, bias, wh):
        xw = (seq_flat @ wi + bias).reshape(b, t, 4 * hid)
        h = jnp.zeros((b, hid), jnp.float32)
        c = jnp.zeros((b, hid), jnp.float32)
        hs = []
        for step in range(t):
            g = xw[:, step, :] + h @ wh
            ig = jax.nn.sigmoid(g[:, :hid])
            fg = jax.nn.sigmoid(g[:, hid:2 * hid])
            gg = jnp.tanh(g[:, 2 * hid:3 * hid])
            og = jax.nn.sigmoid(g[:, 3 * hid:])
            c = fg * c + ig * gg
            h = og * jnp.tanh(c)
            hs.append(h)
        return hs

    hs0 = run(x, wi0_ref[0], b0_ref[0], wh0_ref[0])
    h1in = jnp.stack(hs0, axis=1).reshape(b * t, hid)
    hs1 = run(h1in, wi1_ref[0], b1_ref[0], wh1_ref[0])
    o_ref[0] = hs1[-1] @ fw_ref[0] + fb_ref[0]


def _lstm_fc(x, p, b, t, i):
    hid = 128
    wi0 = p['l0_wih'].transpose(0, 2, 1)
    wh0 = p['l0_whh'].transpose(0, 2, 1)
    b0 = (p['l0_bih'] + p['l0_bhh'])[:, None, :]
    wi1 = p['l1_wih'].transpose(0, 2, 1)
    wh1 = p['l1_whh'].transpose(0, 2, 1)
    b1 = (p['l1_bih'] + p['l1_bhh'])[:, None, :]
    fw = p['fc_w'].transpose(0, 2, 1)
    fb = p['fc_b'][:, None, :]
    body = functools.partial(_lstm_body, b=b, t=t, hid=hid)
    full = lambda a: pl.BlockSpec((1,) + a.shape[1:],
                                  lambda m: (m,) + (0,) * (a.ndim - 1))
    ins = (x, wi0, wh0, b0, wi1, wh1, b1, fw, fb)
    return pl.pallas_call(
        body,
        grid=(_NE,),
        in_specs=[full(a) for a in ins],
        out_specs=pl.BlockSpec((1, b, 2), lambda m: (m, 0, 0)),
        out_shape=jax.ShapeDtypeStruct((_NE, b, 2), jnp.float32),
        compiler_params=_CP(dimension_semantics=("parallel",)),
    )(*ins)


def _asm_body(a_ref, v_ref, ea0_ref, ea1_ref, ev0_ref, ev1_ref,
              oc1_ref, oc2_ref, votes_ref, mat_ref):
    a = a_ref[...]
    v = v_ref[...]
    a0, a1 = a[:, :, 0], a[:, :, 1]
    v0, v1 = v[:, :, 0], v[:, :, 1]
    mat_ref[...] = (ea0_ref[...] @ a0 + ea1_ref[...] @ a1
                    + ev0_ref[...] @ v1 + ev1_ref[...] @ v0)
    pa = (a0 >= a1).astype(jnp.float32)
    pv = (v0 >= v1).astype(jnp.float32)
    votes_ref[...] = (oc1_ref[...] @ (pa + pv)
                      + oc2_ref[...] @ (2.0 - pa - pv))


def _assemble(out_a, out_v):
    bsz = out_a.shape[1]
    consts = tuple(jnp.asarray(m) for m in
                   (_EA0, _EA1, _EV0, _EV1, _OC1, _OC2))
    ins = (out_a, out_v) + consts
    full = lambda a: pl.BlockSpec(a.shape, lambda i: (0,) * a.ndim)
    votes_t, mat_t = pl.pallas_call(
        _asm_body,
        grid=(1,),
        in_specs=[full(a) for a in ins],
        out_specs=[full(jnp.zeros((_NC, bsz))),
                   full(jnp.zeros((2 * _NC * _NC, bsz)))],
        out_shape=[jax.ShapeDtypeStruct((_NC, bsz), jnp.float32),
                   jax.ShapeDtypeStruct((2 * _NC * _NC, bsz), jnp.float32)],
        compiler_params=_CP(dimension_semantics=("arbitrary",)),
    )(*ins)
    return votes_t.T, mat_t.T.reshape(bsz, 2 * _NC, _NC)


def kernel(x_audio, x_video, audio_params, video_params):
    ap, vp = audio_params, video_params
    bsz = x_audio.shape[0]

    pa = lax.conv_general_dilated_patches(
        x_audio.reshape(bsz, 128, 431, 1), (11, 11), (2, 2),
        [(1, 1), (1, 1)], dimension_numbers=('NHWC', 'HWIO', 'NHWC'))
    wa1 = ap['c1w'].reshape(_NE * 16, 121).T
    ba1 = ap['c1b'].reshape(1, _NE * 16)
    a1 = _conv1(pa, wa1, ba1, row_chunks=3)
    a1 = jnp.pad(a1, ((0, 0), (1, 1), (1, 1), (0, 0)))
    a1 = (a1.reshape(bsz, 16, 2, 54, 2, _NE, 16)
          .transpose(5, 0, 1, 3, 2, 4, 6)
          .reshape(_NE, bsz, 16, 54, 64))
    a2 = _member_conv(a1, _w_off(ap['c2w']), ap['c2b'][:, None, :],
                      na=4, nb=4, oh=13, ow=51, nc=2)
    a2 = jnp.pad(a2, ((0, 0), (0, 0), (1, 1), (1, 2), (0, 0)))
    a3 = _member_conv(_s2d(a2), _w_off(ap['c3w']), ap['c3b'][:, None, :],
                      na=3, nb=3, oh=2, ow=12)
    out_a = _lstm_fc(a3.reshape(_NE, bsz, 6, 48), ap, b=bsz, t=6, i=48)

    tlen = x_video.shape[1]
    xv = x_video.reshape((bsz * tlen,) + x_video.shape[2:])
    pv = lax.conv_general_dilated_patches(
        xv.transpose(0, 2, 3, 1), (7, 7), (3, 3),
        [(1, 1), (1, 1)], dimension_numbers=('NHWC', 'HWIO', 'NHWC'))
    wv1 = vp['c1w'].reshape(_NE * 16, 147).T
    bv1 = vp['c1b'].reshape(1, _NE * 16)
    v1 = _conv1(pv, wv1, bv1, row_chunks=1)
    v1 = jnp.pad(v1, ((0, 0), (1, 2), (1, 2), (0, 0)))
    v1 = (v1.reshape(bsz * tlen, 20, 2, 20, 2, _NE, 16)
          .transpose(5, 0, 1, 3, 2, 4, 6)
          .reshape(_NE, bsz * tlen, 20, 20, 64))
    v2 = _member_conv(v1, _w_off(vp['c2w']), vp['c2b'][:, None, :],
                      na=3, nb=3, oh=18, ow=18, nc=8)
    v2 = jnp.pad(v2, ((0, 0), (0, 0), (1, 2), (1, 2), (0, 0)))
    v3 = _member_conv(_s2d(v2), _w_off(vp['c3w']), vp['c3b'][:, None, :],
                      na=3, nb=3, oh=4, ow=4)
    xv_seq = v3.transpose(0, 1, 4, 2, 3).reshape(_NE, bsz, tlen, 128)
    out_v = _lstm_fc(xv_seq, vp, b=bsz, t=tlen, i=128)

    return _assemble(out_a, out_v)
